# baseline (speedup 1.0000x reference)
import functools

import jax
import jax.numpy as jnp
from jax.experimental import pallas as pl
from jax.experimental.pallas import tpu as pltpu

B, SEQ, EMB = 16, 128, 128
HEADS = 4
HD = EMB // HEADS
LAT = 64
E = 16
TOPK = 2
FF = 512
BASE = 10000.0
T = B * SEQ

KTILE = 512
BLK = 256
NB = 32
NCHUNK = 16


def _rmsnorm_rows(x, g, eps=1e-5):
    return x * jax.lax.rsqrt(jnp.mean(x * x, axis=-1, keepdims=True) + eps) * g


def _prep_kernel(x_ref, g1_ref, xn_ref):
    xn_ref[...] = _rmsnorm_rows(x_ref[...], g1_ref[...])


def _latmm_kernel(xf_ref, wl_ref, bl_ref, out_ref):
    i = pl.program_id(0)
    acc = jax.lax.dot_general(
        xf_ref[...], wl_ref[...], (((1,), (0,)), ((), ())),
        preferred_element_type=jnp.float32)

    @pl.when(i == 0)
    def _():
        out_ref[...] = acc + bl_ref[...]

    @pl.when(i > 0)
    def _():
        out_ref[...] += acc


def _attn_kernel(x_ref, xn_ref, lat_ref, wq_ref, bq_ref, wk_ref, bk_ref,
                 wv_ref, bv_ref, wo_ref, bo_ref, g2_ref, wr_ref, br_ref,
                 h_ref, hn_ref, p1_ref, p2_ref, pos1_ref, pos2_ref, meta_ref):
    xn = xn_ref[...]
    lat = lat_ref[...]

    xnb = xn.astype(jnp.bfloat16)
    latb = lat.astype(jnp.bfloat16)
    q = jax.lax.dot_general(xnb, wq_ref[...].astype(jnp.bfloat16),
                            (((2,), (0,)), ((), ())),
                            preferred_element_type=jnp.float32) + bq_ref[...]
    k = jax.lax.dot_general(latb, wk_ref[...].astype(jnp.bfloat16),
                            (((2,), (0,)), ((), ())),
                            preferred_element_type=jnp.float32) + bk_ref[...]
    v = jax.lax.dot_general(latb, wv_ref[...].astype(jnp.bfloat16),
                            (((2,), (0,)), ((), ())),
                            preferred_element_type=jnp.float32) + bv_ref[...]

    lane = jax.lax.broadcasted_iota(jnp.int32, (SEQ, EMB), 1)
    pos = jax.lax.broadcasted_iota(jnp.int32, (SEQ, EMB), 0).astype(jnp.float32)
    pair = (lane % HD) // 2
    inv_freq = jnp.exp(pair.astype(jnp.float32) * (-2.0 / HD) * jnp.log(BASE))
    ang = pos * inv_freq
    c = jnp.cos(ang)[None]
    s = jnp.sin(ang)[None]
    even = (lane % 2) == 0
    q_nxt = jnp.roll(q, -1, axis=2)
    q_prv = jnp.roll(q, 1, axis=2)
    q = jnp.where(even[None], q * c - q_nxt * s, q_prv * s + q * c)

    scale = 1.0 / (HD ** 0.5)
    o_heads = []
    for hd in range(HEADS):
        qh = q[:, :, hd * HD:(hd + 1) * HD]
        kh = k[:, :, hd * HD:(hd + 1) * HD]
        vh = v[:, :, hd * HD:(hd + 1) * HD]
        sc = jax.lax.dot_general(qh, kh, (((2,), (2,)), ((0,), (0,))),
                                 preferred_element_type=jnp.float32) * scale
        sc = sc - jnp.max(sc, axis=-1, keepdims=True)
        w = jnp.exp(sc)
        w = w / jnp.sum(w, axis=-1, keepdims=True)
        o_heads.append(jax.lax.dot_general(
            w, vh, (((2,), (1,)), ((0,), (0,))),
            preferred_element_type=jnp.float32))
    o = jnp.concatenate(o_heads, axis=2)

    h = x_ref[...] + jax.lax.dot_general(
        o.astype(jnp.bfloat16), wo_ref[...].astype(jnp.bfloat16),
        (((2,), (0,)), ((), ())),
        preferred_element_type=jnp.float32) + bo_ref[...]
    h_ref[...] = h

    hn = _rmsnorm_rows(h, g2_ref[...])
    hn2 = hn.reshape(T, EMB)
    hn_ref[...] = hn2

    logits = jax.lax.dot_general(hn2, wr_ref[...], (((1,), (0,)), ((), ())),
                                 preferred_element_type=jnp.float32) + br_ref[...]
    logits = logits - jnp.max(logits, axis=-1, keepdims=True)
    p = jnp.exp(logits)
    p = p / jnp.sum(p, axis=-1, keepdims=True)

    eidx = jax.lax.broadcasted_iota(jnp.int32, (T, E), 1)
    p1 = jnp.max(p, axis=-1, keepdims=True)
    i1 = jnp.min(jnp.where(p == p1, eidx, E), axis=-1, keepdims=True)
    pm = jnp.where(eidx == i1, -jnp.inf, p)
    p2 = jnp.max(pm, axis=-1, keepdims=True)
    i2 = jnp.min(jnp.where(pm == p2, eidx, E), axis=-1, keepdims=True)
    p1_ref[...] = p1
    p2_ref[...] = p2

    oh1 = (eidx == i1)
    oh2 = (eidx == i2)
    C = jnp.where(oh1 | oh2, 1.0, 0.0)
    C3 = C.reshape(NCHUNK, T // NCHUNK, E)
    acc = C3
    k = 1
    while k < T // NCHUNK:
        sh = jnp.concatenate(
            [jnp.zeros((NCHUNK, k, E), jnp.float32), acc[:, :-k, :]], axis=1)
        acc = acc + sh
        k *= 2
    S = acc[:, -1, :]
    sacc = S
    k = 1
    while k < NCHUNK:
        sacc = sacc + jnp.concatenate(
            [jnp.zeros((k, E), jnp.float32), sacc[:-k, :]], axis=0)
        k *= 2
    base_c = sacc - S
    rank = ((acc - C3) + base_c[:, None, :]).reshape(T, E)
    counts = sacc[-1:, :]
    cnt_pad = jnp.ceil(counts * (1.0 / BLK)) * BLK
    bacc = cnt_pad
    k = 1
    while k < E:
        bacc = bacc + jnp.concatenate(
            [jnp.zeros((1, k), jnp.float32), bacc[:, :-k]], axis=1)
        k *= 2
    base_e = bacc - cnt_pad
    slotpos = base_e + rank
    pos1_ref[...] = jnp.sum(jnp.where(oh1, slotpos, 0.0), axis=-1,
                            keepdims=True).astype(jnp.int32)
    pos2_ref[...] = jnp.sum(jnp.where(oh2, slotpos, 0.0), axis=-1,
                            keepdims=True).astype(jnp.int32)

    bidx = jax.lax.broadcasted_iota(jnp.int32, (NB, E), 0).astype(jnp.float32)
    owner = jnp.sum(jnp.where(base_e <= bidx * BLK, 1.0, 0.0), axis=-1,
                    keepdims=True) - 1.0
    used = jnp.sum(cnt_pad, axis=-1, keepdims=True)
    nact = jnp.ceil(used * (1.0 / BLK))
    meta_ref[...] = jnp.concatenate(
        [owner, jnp.broadcast_to(nact, (NB, 1))], axis=1).astype(jnp.int32)


def _moe_kernel(meta_ref, hn_ref, h_ref, p1_ref, p2_ref, pos1_ref, pos2_ref,
                w1_ref, b1_ref, ws_ref, bs_ref, w2_ref, b2_ref, out_ref):
    b = pl.program_id(0)

    @pl.when(b == 0)
    def _():
        out_ref[...] = h_ref[...]

    @pl.when(b < meta_ref[0, 1])
    def _():
        slot = jax.lax.broadcasted_iota(jnp.int32, (T, BLK), 1) + b * BLK
        sel1 = slot == pos1_ref[...]
        sel2 = slot == pos2_ref[...]
        ptb = (jnp.where(sel1, 1.0, 0.0)
               + jnp.where(sel2, 1.0, 0.0)).astype(jnp.bfloat16)
        gtb = jnp.where(sel1, p1_ref[...], 0.0) + jnp.where(sel2, p2_ref[...], 0.0)
        xb = jax.lax.dot_general(ptb, hn_ref[...].astype(jnp.bfloat16),
                                 (((0,), (0,)), ((), ())),
                                 preferred_element_type=jnp.float32)
        h1 = jax.lax.dot_general(xb.astype(jnp.bfloat16),
                                 w1_ref[0].astype(jnp.bfloat16),
                                 (((1,), (0,)), ((), ())),
                                 preferred_element_type=jnp.float32) + b1_ref[0]
        h2 = jax.lax.dot_general(h1.astype(jnp.bfloat16),
                                 ws_ref[0].astype(jnp.bfloat16),
                                 (((1,), (0,)), ((), ())),
                                 preferred_element_type=jnp.float32) + bs_ref[0]
        h2 = jnp.maximum(h2, 0.0)
        eo = jax.lax.dot_general(h2.astype(jnp.bfloat16),
                                 w2_ref[0].astype(jnp.bfloat16),
                                 (((1,), (0,)), ((), ())),
                                 preferred_element_type=jnp.float32) + b2_ref[0]
        out_ref[...] += jax.lax.dot_general(
            gtb.astype(jnp.bfloat16), eo.astype(jnp.bfloat16),
            (((1,), (0,)), ((), ())), preferred_element_type=jnp.float32)


def kernel(x, g1, g2, Wl, bl, Wq, bq, Wk, bk, Wv, bv, Wo, bo, Wr, br,
           W1, b1, Ws, bs, W2, b2):
    xn = pl.pallas_call(
        _prep_kernel,
        out_shape=jax.ShapeDtypeStruct((B, SEQ, EMB), jnp.float32),
    )(x, g1)

    xf = xn.reshape(B, SEQ * EMB)
    nk = (SEQ * EMB) // KTILE
    lat = pl.pallas_call(
        _latmm_kernel,
        grid=(nk,),
        in_specs=[
            pl.BlockSpec((B, KTILE), lambda i: (0, i)),
            pl.BlockSpec((KTILE, LAT * EMB), lambda i: (i, 0)),
            pl.BlockSpec((LAT * EMB,), lambda i: (0,)),
        ],
        out_specs=pl.BlockSpec((B, LAT * EMB), lambda i: (0, 0)),
        out_shape=jax.ShapeDtypeStruct((B, LAT * EMB), jnp.float32),
    )(xf, Wl, bl)

    lat3 = lat.reshape(B, LAT, EMB)
    h, hn, p1, p2, pos1, pos2, meta = pl.pallas_call(
        _attn_kernel,
        out_shape=(
            jax.ShapeDtypeStruct((B, SEQ, EMB), jnp.float32),
            jax.ShapeDtypeStruct((T, EMB), jnp.float32),
            jax.ShapeDtypeStruct((T, 1), jnp.float32),
            jax.ShapeDtypeStruct((T, 1), jnp.float32),
            jax.ShapeDtypeStruct((T, 1), jnp.int32),
            jax.ShapeDtypeStruct((T, 1), jnp.int32),
            jax.ShapeDtypeStruct((NB, 2), jnp.int32),
        ),
    )(x, xn, lat3, Wq, bq, Wk, bk, Wv, bv, Wo, bo, g2, Wr, br)

    out = pl.pallas_call(
        _moe_kernel,
        grid_spec=pltpu.PrefetchScalarGridSpec(
            num_scalar_prefetch=1,
            grid=(NB,),
            in_specs=[
                pl.BlockSpec((T, EMB), lambda b, m: (0, 0)),
                pl.BlockSpec((T, EMB), lambda b, m: (0, 0)),
                pl.BlockSpec((T, 1), lambda b, m: (0, 0)),
                pl.BlockSpec((T, 1), lambda b, m: (0, 0)),
                pl.BlockSpec((T, 1), lambda b, m: (0, 0)),
                pl.BlockSpec((T, 1), lambda b, m: (0, 0)),
                pl.BlockSpec((1, EMB, FF), lambda b, m: (m[b, 0], 0, 0)),
                pl.BlockSpec((1, 1, FF), lambda b, m: (m[b, 0], 0, 0)),
                pl.BlockSpec((1, FF, FF), lambda b, m: (m[b, 0], 0, 0)),
                pl.BlockSpec((1, 1, FF), lambda b, m: (m[b, 0], 0, 0)),
                pl.BlockSpec((1, FF, EMB), lambda b, m: (m[b, 0], 0, 0)),
                pl.BlockSpec((1, 1, EMB), lambda b, m: (m[b, 0], 0, 0)),
            ],
            out_specs=pl.BlockSpec((T, EMB), lambda b, m: (0, 0)),
        ),
        out_shape=jax.ShapeDtypeStruct((T, EMB), jnp.float32),
    )(meta, hn, h.reshape(T, EMB), p1, p2, pos1, pos2,
      W1, b1.reshape(E, 1, FF), Ws, bs.reshape(E, 1, FF), W2,
      b2.reshape(E, 1, EMB))

    return out.reshape(B, SEQ, EMB)


# speedup vs baseline: 1.0733x; 1.0733x over previous
import jax
import jax.numpy as jnp
from jax.experimental import pallas as pl
from jax.experimental.pallas import tpu as pltpu

B, SEQ, EMB = 16, 128, 128
HEADS = 4
HD = EMB // HEADS
LAT = 64
E = 16
TOPK = 2
FF = 512
BASE = 10000.0
T = B * SEQ

KTILE = 512
KSEQ = KTILE // EMB


def _rmsnorm_rows(x, g, eps=1e-5):
    return x * jax.lax.rsqrt(jnp.mean(x * x, axis=-1, keepdims=True) + eps) * g


def _latmm_kernel(x_ref, g1_ref, wl_ref, bl_ref, out_ref):
    i = pl.program_id(0)
    xn = _rmsnorm_rows(x_ref[:, 0], g1_ref[...])
    xf = xn.reshape(B, KTILE)
    acc = jax.lax.dot_general(
        xf, wl_ref[...], (((1,), (0,)), ((), ())),
        preferred_element_type=jnp.float32)

    @pl.when(i == 0)
    def _():
        out_ref[...] = acc + bl_ref[...]

    @pl.when(i > 0)
    def _():
        out_ref[...] += acc


def _attn_kernel(x_ref, g1_ref, lat_ref, wq_ref, bq_ref, wk_ref, bk_ref,
                 wv_ref, bv_ref, wo_ref, bo_ref, g2_ref, wr_ref, br_ref,
                 h_ref, hn_ref, gates_ref):
    xn = _rmsnorm_rows(x_ref[...], g1_ref[...])
    lat = lat_ref[...]

    xnb = xn.astype(jnp.bfloat16)
    latb = lat.astype(jnp.bfloat16)
    q = jax.lax.dot_general(xnb, wq_ref[...].astype(jnp.bfloat16),
                            (((2,), (0,)), ((), ())),
                            preferred_element_type=jnp.float32) + bq_ref[...]
    k = jax.lax.dot_general(latb, wk_ref[...].astype(jnp.bfloat16),
                            (((2,), (0,)), ((), ())),
                            preferred_element_type=jnp.float32) + bk_ref[...]
    v = jax.lax.dot_general(latb, wv_ref[...].astype(jnp.bfloat16),
                            (((2,), (0,)), ((), ())),
                            preferred_element_type=jnp.float32) + bv_ref[...]

    lane = jax.lax.broadcasted_iota(jnp.int32, (SEQ, EMB), 1)
    pos = jax.lax.broadcasted_iota(jnp.int32, (SEQ, EMB), 0).astype(jnp.float32)
    pair = (lane % HD) // 2
    inv_freq = jnp.exp(pair.astype(jnp.float32) * (-2.0 / HD) * jnp.log(BASE))
    ang = pos * inv_freq
    c = jnp.cos(ang)[None]
    s = jnp.sin(ang)[None]
    even = (lane % 2) == 0
    q_nxt = jnp.roll(q, -1, axis=2)
    q_prv = jnp.roll(q, 1, axis=2)
    q = jnp.where(even[None], q * c - q_nxt * s, q_prv * s + q * c)

    scale = 1.0 / (HD ** 0.5)
    o_heads = []
    for hd in range(HEADS):
        qh = q[:, :, hd * HD:(hd + 1) * HD]
        kh = k[:, :, hd * HD:(hd + 1) * HD]
        vh = v[:, :, hd * HD:(hd + 1) * HD]
        sc = jax.lax.dot_general(qh, kh, (((2,), (2,)), ((0,), (0,))),
                                 preferred_element_type=jnp.float32) * scale
        sc = sc - jnp.max(sc, axis=-1, keepdims=True)
        w = jnp.exp(sc)
        w = w / jnp.sum(w, axis=-1, keepdims=True)
        o_heads.append(jax.lax.dot_general(
            w, vh, (((2,), (1,)), ((0,), (0,))),
            preferred_element_type=jnp.float32))
    o = jnp.concatenate(o_heads, axis=2)

    h = x_ref[...] + jax.lax.dot_general(
        o.astype(jnp.bfloat16), wo_ref[...].astype(jnp.bfloat16),
        (((2,), (0,)), ((), ())),
        preferred_element_type=jnp.float32) + bo_ref[...]
    h_ref[...] = h

    hn = _rmsnorm_rows(h, g2_ref[...])
    hn2 = hn.reshape(T, EMB)
    hn_ref[...] = hn2

    logits = jax.lax.dot_general(hn2, wr_ref[...], (((1,), (0,)), ((), ())),
                                 preferred_element_type=jnp.float32) + br_ref[...]
    logits = logits - jnp.max(logits, axis=-1, keepdims=True)
    p = jnp.exp(logits)
    p = p / jnp.sum(p, axis=-1, keepdims=True)

    eidx = jax.lax.broadcasted_iota(jnp.int32, (T, E), 1)
    p1 = jnp.max(p, axis=-1, keepdims=True)
    i1 = jnp.min(jnp.where(p == p1, eidx, E), axis=-1, keepdims=True)
    pm = jnp.where(eidx == i1, -jnp.inf, p)
    p2 = jnp.max(pm, axis=-1, keepdims=True)
    i2 = jnp.min(jnp.where(pm == p2, eidx, E), axis=-1, keepdims=True)
    gates_ref[...] = jnp.where(eidx == i1, p1, 0.0) + jnp.where(eidx == i2, p2, 0.0)


def _moe_kernel(hn_ref, gates_ref, h_ref, w1_ref, b1_ref, ws_ref, bs_ref,
                w2_ref, b2_ref, out_ref):
    e = pl.program_id(0)
    hn = hn_ref[...].astype(jnp.bfloat16)
    h1 = jax.lax.dot_general(hn, w1_ref[0].astype(jnp.bfloat16),
                             (((1,), (0,)), ((), ())),
                             preferred_element_type=jnp.float32) + b1_ref[0]
    h2 = jax.lax.dot_general(h1.astype(jnp.bfloat16),
                             ws_ref[0].astype(jnp.bfloat16),
                             (((1,), (0,)), ((), ())),
                             preferred_element_type=jnp.float32) + bs_ref[0]
    h2 = jnp.maximum(h2, 0.0)
    eo = jax.lax.dot_general(h2.astype(jnp.bfloat16),
                             w2_ref[0].astype(jnp.bfloat16),
                             (((1,), (0,)), ((), ())),
                             preferred_element_type=jnp.float32) + b2_ref[0]
    eidx = jax.lax.broadcasted_iota(jnp.int32, (T, E), 1)
    g = jnp.sum(jnp.where(eidx == e, gates_ref[...], 0.0), axis=-1,
                keepdims=True)

    @pl.when(e == 0)
    def _():
        out_ref[...] = h_ref[...] + g * eo

    @pl.when(e > 0)
    def _():
        out_ref[...] += g * eo


def kernel(x, g1, g2, Wl, bl, Wq, bq, Wk, bk, Wv, bv, Wo, bo, Wr, br,
           W1, b1, Ws, bs, W2, b2):
    nk = (SEQ * EMB) // KTILE
    lat = pl.pallas_call(
        _latmm_kernel,
        grid=(nk,),
        in_specs=[
            pl.BlockSpec((B, 1, KSEQ, EMB), lambda i: (0, i, 0, 0)),
            pl.BlockSpec((EMB,), lambda i: (0,)),
            pl.BlockSpec((KTILE, LAT * EMB), lambda i: (i, 0)),
            pl.BlockSpec((LAT * EMB,), lambda i: (0,)),
        ],
        out_specs=pl.BlockSpec((B, LAT * EMB), lambda i: (0, 0)),
        out_shape=jax.ShapeDtypeStruct((B, LAT * EMB), jnp.float32),
    )(x.reshape(B, nk, KSEQ, EMB), g1, Wl, bl)

    lat3 = lat.reshape(B, LAT, EMB)
    h, hn, gates = pl.pallas_call(
        _attn_kernel,
        out_shape=(
            jax.ShapeDtypeStruct((B, SEQ, EMB), jnp.float32),
            jax.ShapeDtypeStruct((T, EMB), jnp.float32),
            jax.ShapeDtypeStruct((T, E), jnp.float32),
        ),
    )(x, g1, lat3, Wq, bq, Wk, bk, Wv, bv, Wo, bo, g2, Wr, br)

    out = pl.pallas_call(
        _moe_kernel,
        grid=(E,),
        in_specs=[
            pl.BlockSpec((T, EMB), lambda e: (0, 0)),
            pl.BlockSpec((T, E), lambda e: (0, 0)),
            pl.BlockSpec((T, EMB), lambda e: (0, 0)),
            pl.BlockSpec((1, EMB, FF), lambda e: (e, 0, 0)),
            pl.BlockSpec((1, 1, FF), lambda e: (e, 0, 0)),
            pl.BlockSpec((1, FF, FF), lambda e: (e, 0, 0)),
            pl.BlockSpec((1, 1, FF), lambda e: (e, 0, 0)),
            pl.BlockSpec((1, FF, EMB), lambda e: (e, 0, 0)),
            pl.BlockSpec((1, 1, EMB), lambda e: (e, 0, 0)),
        ],
        out_specs=pl.BlockSpec((T, EMB), lambda e: (0, 0)),
        out_shape=jax.ShapeDtypeStruct((T, EMB), jnp.float32),
    )(hn, gates, h.reshape(T, EMB), W1, b1.reshape(E, 1, FF), Ws,
      bs.reshape(E, 1, FF), W2, b2.reshape(E, 1, EMB))

    return out.reshape(B, SEQ, EMB)
